# Initial kernel scaffold; baseline (speedup 1.0000x reference)
import jax
import jax.numpy as jnp
from jax.experimental import pallas as pl


def kernel(data, x, edge_index, W_e1, b_e1, W_e2, b_e2, W_d1, b_d1, W_d2, b_d2):
    raise NotImplementedError("write your pallas kernel here")


# speedup vs baseline: 5.3123x; 5.3123x over previous
import functools

import jax
import jax.numpy as jnp
from jax import lax
from jax.experimental import pallas as pl
from jax.experimental.pallas import tpu as pltpu
from jax.experimental.pallas import tpu_sc as plsc

N = 10000
D = 256
HALF = 5000
HALFP = 5120
NROWS = 2 * HALFP
HH = 128
NTRASH = 128
ACC_ROWS = 2 * (HALFP + NTRASH)
CH = 128
RPT = 2 * HALFP // 16
CH16 = CH // 16
_BR = 1024


def _make_prop(epad, gather):
    e_per_tile = epad // 16
    n_chunks = e_per_tile // CH
    mesh = plsc.VectorSubcoreMesh(core_axis_name="c", subcore_axis_name="s")

    scratch = [
        pltpu.VMEM((CH,), jnp.int32),
        pltpu.VMEM((CH,), jnp.int32),
        pltpu.VMEM((CH,), jnp.int32),
        pltpu.VMEM((CH, HH), jnp.float32),
        pltpu.VMEM((CH, HH), jnp.float32),
        pltpu.VMEM_SHARED((ACC_ROWS, HH), jnp.float32),
        pltpu.SemaphoreType.DMA,
        pltpu.SemaphoreType.DMA,
    ]
    if gather:
        scratch = [pltpu.VMEM((CH,), jnp.int32),
                   pltpu.VMEM((CH,), jnp.int32),
                   pltpu.VMEM((CH,), jnp.int32),
                   ] + scratch

    @functools.partial(pl.kernel, mesh=mesh,
                       out_type=jax.ShapeDtypeStruct((2 * NROWS, HH),
                                                     jnp.float32),
                       scratch_types=scratch)
    def prop(*refs):
        if gather:
            (g_hbm, src_hbm, dst_hbm, out_hbm, srcv, srcA, srcB,
             dstv, dstA, dstB, rowsA, rowsB, acc, semA, semB) = refs
        else:
            (ones_hbm, dst_hbm, out_hbm,
             dstv, dstA, dstB, rowsA, rowsB, acc, semA, semB) = refs
        c = lax.axis_index("c")
        s = lax.axis_index("s")
        lo = c * HALF
        if gather:
            pltpu.sync_copy(g_hbm.at[pl.ds(c * (2 * HALFP) + s * RPT, RPT)],
                            acc.at[pl.ds(s * RPT, RPT)])
        else:
            pltpu.sync_copy(ones_hbm, acc.at[pl.ds(s * RPT, RPT)])
            pltpu.sync_copy(ones_hbm.at[pl.ds(0, CH)], rowsA)
            pltpu.sync_copy(ones_hbm.at[pl.ds(0, CH)], rowsB)
        plsc.subcore_barrier()

        base = s * e_per_tile

        def chunk(j, carry):
            eb = pl.multiple_of(base + j * CH, CH)
            pltpu.sync_copy(dst_hbm.at[pl.ds(eb, CH)], dstv)
            if gather:
                pltpu.sync_copy(src_hbm.at[pl.ds(eb, CH)], srcv)
            for k in range(CH16):
                sl = pl.ds(k * 16, 16)
                d16 = dstv[sl]
                l16 = d16 - lo
                ok = (l16 >= 0) & (l16 < HALF)
                lcl = jnp.where(ok, l16, HALFP + (d16 & (NTRASH - 1)))
                dstA[sl] = 2 * lcl
                dstB[sl] = 2 * lcl + 1
                if gather:
                    s16 = srcv[sl]
                    sph = jnp.where(s16 >= HALF, s16 + (HALFP - HALF), s16)
                    srcA[sl] = 2 * sph
                    srcB[sl] = 2 * sph + 1
            if gather:
                cpA = pltpu.async_copy(g_hbm.at[srcA], rowsA, semA)
                cpB = pltpu.async_copy(g_hbm.at[srcB], rowsB, semB)
                cpA.wait()
                cpB.wait()
            aA = pltpu.async_copy(rowsA, acc.at[dstA], semA, add=True)
            aB = pltpu.async_copy(rowsB, acc.at[dstB], semB, add=True)
            aA.wait()
            aB.wait()
            return carry

        lax.fori_loop(0, n_chunks, chunk, 0)
        plsc.subcore_barrier()
        pltpu.sync_copy(acc.at[pl.ds(s * RPT, RPT)],
                        out_hbm.at[pl.ds(c * (2 * HALFP) + s * RPT, RPT)])

    return prop


def _mm_first():
    def body(x_ref, dis_ref, w_ref, out_ref):
        o = jnp.dot(x_ref[...], w_ref[...],
                    preferred_element_type=jnp.float32)
        out_ref[...] = o * dis_ref[...]

    return pl.pallas_call(
        body,
        grid=(NROWS // _BR,),
        in_specs=[
            pl.BlockSpec((_BR, D), lambda i: (i, 0)),
            pl.BlockSpec((_BR, D), lambda i: (i, 0)),
            pl.BlockSpec((D, D), lambda i: (0, 0)),
        ],
        out_specs=pl.BlockSpec((_BR, D), lambda i: (i, 0)),
        out_shape=jax.ShapeDtypeStruct((NROWS, D), jnp.float32),
    )


def _mm_mid(relu):
    def body(s_ref, dis_ref, b_ref, w_ref, out_ref):
        t = s_ref[...] * dis_ref[...] + b_ref[...]
        if relu:
            t = jnp.maximum(t, 0.0)
        o = jnp.dot(t, w_ref[...], preferred_element_type=jnp.float32)
        out_ref[...] = o * dis_ref[...]

    return pl.pallas_call(
        body,
        grid=(NROWS // _BR,),
        in_specs=[
            pl.BlockSpec((_BR, D), lambda i: (i, 0)),
            pl.BlockSpec((_BR, D), lambda i: (i, 0)),
            pl.BlockSpec((1, D), lambda i: (0, 0)),
            pl.BlockSpec((D, D), lambda i: (0, 0)),
        ],
        out_specs=pl.BlockSpec((_BR, D), lambda i: (i, 0)),
        out_shape=jax.ShapeDtypeStruct((NROWS, D), jnp.float32),
    )


def _rsqrt_body(deg_ref, out_ref):
    out_ref[...] = lax.rsqrt(deg_ref[...])


_rsqrt_kernel = pl.pallas_call(
    _rsqrt_body,
    grid=(NROWS // _BR,),
    in_specs=[pl.BlockSpec((_BR, D), lambda i: (i, 0))],
    out_specs=pl.BlockSpec((_BR, D), lambda i: (i, 0)),
    out_shape=jax.ShapeDtypeStruct((NROWS, D), jnp.float32),
)


def _finish_body(s_ref, dis_ref, b_ref, out_ref):
    out_ref[...] = s_ref[...] * dis_ref[...] + b_ref[...]


_finish = pl.pallas_call(
    _finish_body,
    grid=(NROWS // _BR,),
    in_specs=[
        pl.BlockSpec((_BR, D), lambda i: (i, 0)),
        pl.BlockSpec((_BR, D), lambda i: (i, 0)),
        pl.BlockSpec((1, D), lambda i: (0, 0)),
    ],
    out_specs=pl.BlockSpec((_BR, D), lambda i: (i, 0)),
    out_shape=jax.ShapeDtypeStruct((NROWS, D), jnp.float32),
)


def kernel(data, x, edge_index, W_e1, b_e1, W_e2, b_e2, W_d1, b_d1, W_d2, b_d2):
    src = edge_index[0]
    dst = edge_index[1]
    e = src.shape[0]
    epad = -(-e // (16 * CH)) * (16 * CH)
    if epad != e:
        pad = jnp.arange(epad - e, dtype=jnp.int32)
        src_p = jnp.concatenate([src, (pad * 2003) % N])
        dst_p = jnp.concatenate([dst, jnp.full((epad - e,), N, jnp.int32)])
    else:
        src_p, dst_p = src, dst

    prop_deg = _make_prop(epad, gather=False)
    prop = _make_prop(epad, gather=True)

    def run_prop(g):
        out2 = prop(g.reshape(2 * NROWS, HH), src_p, dst_p)
        return out2.reshape(NROWS, D)

    ones_blk = jnp.ones((RPT, HH), jnp.float32)
    deg2 = prop_deg(ones_blk, dst_p)
    dis = _rsqrt_kernel(deg2.reshape(NROWS, D))

    zpad = jnp.zeros((HALFP - HALF, D), jnp.float32)
    xp = jnp.concatenate([x[:HALF], zpad, x[HALF:], zpad], axis=0)
    g1 = _mm_first()(xp, dis, W_e1)
    s1 = run_prop(g1)
    g2 = _mm_mid(True)(s1, dis, b_e1.reshape(1, D), W_e2)
    s2 = run_prop(g2)
    g3 = _mm_mid(False)(s2, dis, b_e2.reshape(1, D), W_d1)
    s3 = run_prop(g3)
    g4 = _mm_mid(True)(s3, dis, b_d1.reshape(1, D), W_d2)
    s4 = run_prop(g4)
    y = _finish(s4, dis, b_d2.reshape(1, D))

    x_ = jnp.concatenate([y[:HALF], y[HALFP:HALFP + HALF]], axis=0)
    return (x_, 1, 1)


# speedup vs baseline: 7.2045x; 1.3562x over previous
import functools

import jax
import jax.numpy as jnp
from jax import lax
from jax.experimental import pallas as pl
from jax.experimental.pallas import tpu as pltpu
from jax.experimental.pallas import tpu_sc as plsc

N = 10000
D = 256
HALF = 5000
HALFP = 5120
NROWS = 2 * HALFP
HH = 128
NTRASH = 128
ACC_ROWS = 2 * (HALFP + NTRASH)
CH = 80
RPT = 2 * HALFP // 16
CH16 = CH // 16
_BR = 1024


def _make_prop(epad, gather):
    e_per_tile = epad // 16
    n_chunks = e_per_tile // CH
    n_pairs = n_chunks // 2
    mesh = plsc.VectorSubcoreMesh(core_axis_name="c", subcore_axis_name="s")

    def _vec2(shape, dt):
        return [pltpu.VMEM(shape, dt), pltpu.VMEM(shape, dt)]

    scratch = (
        _vec2((CH,), jnp.int32)
        + _vec2((CH,), jnp.int32)
        + _vec2((CH,), jnp.int32)
        + _vec2((CH, HH), jnp.float32)
        + _vec2((CH, HH), jnp.float32)
        + [pltpu.VMEM_SHARED((ACC_ROWS, HH), jnp.float32)]
        + [pltpu.SemaphoreType.DMA] * 4
    )
    if gather:
        scratch = (
            _vec2((CH,), jnp.int32)
            + _vec2((CH,), jnp.int32)
            + _vec2((CH,), jnp.int32)
            + scratch
            + [pltpu.SemaphoreType.DMA] * 2
        )

    @functools.partial(pl.kernel, mesh=mesh,
                       out_type=jax.ShapeDtypeStruct((2 * NROWS, HH),
                                                     jnp.float32),
                       scratch_types=scratch)
    def prop(*refs):
        if gather:
            (g_hbm, src_hbm, dst_hbm, out_hbm,
             srcv0, srcv1, srcA0, srcA1, srcB0, srcB1,
             dstv0, dstv1, dstA0, dstA1, dstB0, dstB1,
             rowsA0, rowsA1, rowsB0, rowsB1, acc,
             isem0, isem1, ssem0, ssem1, gsem0, gsem1) = refs
            srcv, srcA, srcB = (srcv0, srcv1), (srcA0, srcA1), (srcB0, srcB1)
            gsem = (gsem0, gsem1)
        else:
            (ones_hbm, dst_hbm, out_hbm,
             dstv0, dstv1, dstA0, dstA1, dstB0, dstB1,
             rowsA0, rowsA1, rowsB0, rowsB1, acc,
             isem0, isem1, ssem0, ssem1) = refs
        dstv, dstA, dstB = (dstv0, dstv1), (dstA0, dstA1), (dstB0, dstB1)
        rowsA, rowsB = (rowsA0, rowsA1), (rowsB0, rowsB1)
        isem, ssem = (isem0, isem1), (ssem0, ssem1)
        c = lax.axis_index("c")
        s = lax.axis_index("s")
        lo = c * HALF
        base = s * e_per_tile

        def load_idx(cj, b):
            eb = pl.multiple_of(base + lax.min(cj, n_chunks - 1) * CH, CH)
            pltpu.async_copy(dst_hbm.at[pl.ds(eb, CH)], dstv[b], isem[b])
            if gather:
                pltpu.async_copy(src_hbm.at[pl.ds(eb, CH)], srcv[b], isem[b])

        def wait_idx(b):
            pltpu.make_async_copy(dst_hbm.at[pl.ds(0, CH)], dstv[b],
                                  isem[b]).wait()
            if gather:
                pltpu.make_async_copy(src_hbm.at[pl.ds(0, CH)], srcv[b],
                                      isem[b]).wait()

        def compute_idx(b):
            for k in range(CH16):
                sl = pl.ds(k * 16, 16)
                d16 = dstv[b][sl]
                l16 = d16 - lo
                ok = (l16 >= 0) & (l16 < HALF)
                lcl = jnp.where(ok, l16, HALFP + (d16 & (NTRASH - 1)))
                dstA[b][sl] = 2 * lcl
                dstB[b][sl] = 2 * lcl + 1
                if gather:
                    s16 = srcv[b][sl]
                    sph = jnp.where(s16 >= HALF, s16 + (HALFP - HALF), s16)
                    srcA[b][sl] = 2 * sph
                    srcB[b][sl] = 2 * sph + 1

        def fire_gather(b):
            pltpu.async_copy(g_hbm.at[srcA[b]], rowsA[b], gsem[b])
            pltpu.async_copy(g_hbm.at[srcB[b]], rowsB[b], gsem[b])

        def wait_gather(b):
            pltpu.make_async_copy(g_hbm.at[pl.ds(0, CH)], rowsA[b],
                                  gsem[b]).wait()
            pltpu.make_async_copy(g_hbm.at[pl.ds(0, CH)], rowsB[b],
                                  gsem[b]).wait()

        def fire_scatter(b):
            pltpu.async_copy(rowsA[b], acc.at[dstA[b]], ssem[b], add=True)
            pltpu.async_copy(rowsB[b], acc.at[dstB[b]], ssem[b], add=True)

        def wait_scatter(b):
            pltpu.make_async_copy(rowsA[b], acc.at[pl.ds(0, CH)],
                                  ssem[b]).wait()
            pltpu.make_async_copy(rowsB[b], acc.at[pl.ds(0, CH)],
                                  ssem[b]).wait()

        if gather:
            pltpu.sync_copy(g_hbm.at[pl.ds(c * (2 * HALFP) + s * RPT, RPT)],
                            acc.at[pl.ds(s * RPT, RPT)])
        else:
            pltpu.sync_copy(ones_hbm, acc.at[pl.ds(s * RPT, RPT)])
            pltpu.sync_copy(ones_hbm.at[pl.ds(0, CH)], rowsA[0])
            pltpu.sync_copy(ones_hbm.at[pl.ds(0, CH)], rowsB[0])
            pltpu.sync_copy(ones_hbm.at[pl.ds(0, CH)], rowsA[1])
            pltpu.sync_copy(ones_hbm.at[pl.ds(0, CH)], rowsB[1])
        plsc.subcore_barrier()

        if gather:
            load_idx(0, 0)
            load_idx(1, 1)
            wait_idx(0)
            compute_idx(0)
            fire_gather(0)
            wait_idx(1)
            compute_idx(1)
            wait_gather(0)
            fire_scatter(0)
            fire_gather(1)
            load_idx(2, 0)
            wait_idx(0)
            wait_scatter(0)
            compute_idx(0)
            wait_gather(1)
            fire_scatter(1)
            fire_gather(0)
            load_idx(3, 1)

            def pair(g, carry):
                wait_idx(1)
                wait_scatter(1)
                compute_idx(1)
                wait_gather(0)
                fire_scatter(0)
                fire_gather(1)
                load_idx(2 * g + 2, 0)
                wait_idx(0)
                wait_scatter(0)
                compute_idx(0)
                wait_gather(1)
                fire_scatter(1)
                fire_gather(0)
                load_idx(2 * g + 3, 1)
                return carry

            lax.fori_loop(1, n_pairs, pair, 0)
            wait_idx(1)
            wait_scatter(1)
            wait_gather(0)
        else:
            load_idx(0, 0)
            load_idx(1, 1)
            wait_idx(0)
            compute_idx(0)
            fire_scatter(0)
            load_idx(2, 0)
            wait_idx(1)
            compute_idx(1)
            fire_scatter(1)
            load_idx(3, 1)

            def pair(g, carry):
                wait_idx(0)
                wait_scatter(0)
                compute_idx(0)
                fire_scatter(0)
                load_idx(2 * g + 2, 0)
                wait_idx(1)
                wait_scatter(1)
                compute_idx(1)
                fire_scatter(1)
                load_idx(2 * g + 3, 1)
                return carry

            lax.fori_loop(1, n_pairs, pair, 0)
            wait_idx(0)
            wait_idx(1)
            wait_scatter(0)
            wait_scatter(1)
        plsc.subcore_barrier()
        pltpu.sync_copy(acc.at[pl.ds(s * RPT, RPT)],
                        out_hbm.at[pl.ds(c * (2 * HALFP) + s * RPT, RPT)])

    return prop


def _mm_first():
    def body(x_ref, dis_ref, w_ref, out_ref):
        o = jnp.dot(x_ref[...], w_ref[...],
                    preferred_element_type=jnp.float32)
        out_ref[...] = o * dis_ref[...]

    return pl.pallas_call(
        body,
        grid=(NROWS // _BR,),
        in_specs=[
            pl.BlockSpec((_BR, D), lambda i: (i, 0)),
            pl.BlockSpec((_BR, D), lambda i: (i, 0)),
            pl.BlockSpec((D, D), lambda i: (0, 0)),
        ],
        out_specs=pl.BlockSpec((_BR, D), lambda i: (i, 0)),
        out_shape=jax.ShapeDtypeStruct((NROWS, D), jnp.float32),
    )


def _mm_mid(relu):
    def body(s_ref, dis_ref, b_ref, w_ref, out_ref):
        t = s_ref[...] * dis_ref[...] + b_ref[...]
        if relu:
            t = jnp.maximum(t, 0.0)
        o = jnp.dot(t, w_ref[...], preferred_element_type=jnp.float32)
        out_ref[...] = o * dis_ref[...]

    return pl.pallas_call(
        body,
        grid=(NROWS // _BR,),
        in_specs=[
            pl.BlockSpec((_BR, D), lambda i: (i, 0)),
            pl.BlockSpec((_BR, D), lambda i: (i, 0)),
            pl.BlockSpec((1, D), lambda i: (0, 0)),
            pl.BlockSpec((D, D), lambda i: (0, 0)),
        ],
        out_specs=pl.BlockSpec((_BR, D), lambda i: (i, 0)),
        out_shape=jax.ShapeDtypeStruct((NROWS, D), jnp.float32),
    )


def _rsqrt_body(deg_ref, out_ref):
    out_ref[...] = lax.rsqrt(deg_ref[...])


_rsqrt_kernel = pl.pallas_call(
    _rsqrt_body,
    grid=(NROWS // _BR,),
    in_specs=[pl.BlockSpec((_BR, D), lambda i: (i, 0))],
    out_specs=pl.BlockSpec((_BR, D), lambda i: (i, 0)),
    out_shape=jax.ShapeDtypeStruct((NROWS, D), jnp.float32),
)


def _finish_body(s_ref, dis_ref, b_ref, out_ref):
    out_ref[...] = s_ref[...] * dis_ref[...] + b_ref[...]


_finish = pl.pallas_call(
    _finish_body,
    grid=(NROWS // _BR,),
    in_specs=[
        pl.BlockSpec((_BR, D), lambda i: (i, 0)),
        pl.BlockSpec((_BR, D), lambda i: (i, 0)),
        pl.BlockSpec((1, D), lambda i: (0, 0)),
    ],
    out_specs=pl.BlockSpec((_BR, D), lambda i: (i, 0)),
    out_shape=jax.ShapeDtypeStruct((NROWS, D), jnp.float32),
)


def kernel(data, x, edge_index, W_e1, b_e1, W_e2, b_e2, W_d1, b_d1, W_d2, b_d2):
    src = edge_index[0]
    dst = edge_index[1]
    e = src.shape[0]
    epad = -(-e // (16 * CH * 2)) * (16 * CH * 2)
    if epad != e:
        pad = jnp.arange(epad - e, dtype=jnp.int32)
        src_p = jnp.concatenate([src, (pad * 2003) % N])
        dst_p = jnp.concatenate([dst, jnp.full((epad - e,), N, jnp.int32)])
    else:
        src_p, dst_p = src, dst

    prop_deg = _make_prop(epad, gather=False)
    prop = _make_prop(epad, gather=True)

    def run_prop(g):
        out2 = prop(g.reshape(2 * NROWS, HH), src_p, dst_p)
        return out2.reshape(NROWS, D)

    ones_blk = jnp.ones((RPT, HH), jnp.float32)
    deg2 = prop_deg(ones_blk, dst_p)
    dis = _rsqrt_kernel(deg2.reshape(NROWS, D))

    zpad = jnp.zeros((HALFP - HALF, D), jnp.float32)
    xp = jnp.concatenate([x[:HALF], zpad, x[HALF:], zpad], axis=0)
    g1 = _mm_first()(xp, dis, W_e1)
    s1 = run_prop(g1)
    g2 = _mm_mid(True)(s1, dis, b_e1.reshape(1, D), W_e2)
    s2 = run_prop(g2)
    g3 = _mm_mid(False)(s2, dis, b_e2.reshape(1, D), W_d1)
    s3 = run_prop(g3)
    g4 = _mm_mid(True)(s3, dis, b_d1.reshape(1, D), W_d2)
    s4 = run_prop(g4)
    y = _finish(s4, dis, b_d2.reshape(1, D))

    x_ = jnp.concatenate([y[:HALF], y[HALFP:HALFP + HALF]], axis=0)
    return (x_, 1, 1)


# speedup vs baseline: 8.0972x; 1.1239x over previous
import functools

import jax
import jax.numpy as jnp
from jax import lax
from jax.experimental import pallas as pl
from jax.experimental.pallas import tpu as pltpu
from jax.experimental.pallas import tpu_sc as plsc

N = 10000
D = 256
HALF = 5000
HALFP = 5120
NROWS = 2 * HALFP
HH = 128
SENT = -1
ACC_ROWS = 2 * HALFP
CH = 80
RPT = 2 * HALFP // 16
CH16 = CH // 16
_BR = 1024


def _make_prop(epad, gather):
    e_per_tile = epad // 16
    n_chunks = e_per_tile // CH
    n_pairs = n_chunks // 2
    mesh = plsc.VectorSubcoreMesh(core_axis_name="c", subcore_axis_name="s")

    def _vec2(shape, dt):
        return [pltpu.VMEM(shape, dt), pltpu.VMEM(shape, dt)]

    scratch = (
        _vec2((CH,), jnp.int32)
        + _vec2((CH,), jnp.int32)
        + _vec2((CH,), jnp.int32)
        + _vec2((CH, HH), jnp.float32)
        + _vec2((CH, HH), jnp.float32)
        + [pltpu.VMEM_SHARED((ACC_ROWS, HH), jnp.float32)]
        + [pltpu.SemaphoreType.DMA] * 4
    )
    if gather:
        scratch = (
            _vec2((CH,), jnp.int32)
            + _vec2((CH,), jnp.int32)
            + _vec2((CH,), jnp.int32)
            + scratch
            + [pltpu.SemaphoreType.DMA] * 2
        )

    @functools.partial(pl.kernel, mesh=mesh,
                       out_type=jax.ShapeDtypeStruct((2 * NROWS, HH),
                                                     jnp.float32),
                       scratch_types=scratch)
    def prop(*refs):
        if gather:
            (g_hbm, src_hbm, dst_hbm, out_hbm,
             srcv0, srcv1, srcA0, srcA1, srcB0, srcB1,
             dstv0, dstv1, dstA0, dstA1, dstB0, dstB1,
             rowsA0, rowsA1, rowsB0, rowsB1, acc,
             isem0, isem1, ssem0, ssem1, gsem0, gsem1) = refs
            srcv, srcA, srcB = (srcv0, srcv1), (srcA0, srcA1), (srcB0, srcB1)
            gsem = (gsem0, gsem1)
        else:
            (ones_hbm, dst_hbm, out_hbm,
             dstv0, dstv1, dstA0, dstA1, dstB0, dstB1,
             rowsA0, rowsA1, rowsB0, rowsB1, acc,
             isem0, isem1, ssem0, ssem1) = refs
        dstv, dstA, dstB = (dstv0, dstv1), (dstA0, dstA1), (dstB0, dstB1)
        rowsA, rowsB = (rowsA0, rowsA1), (rowsB0, rowsB1)
        isem, ssem = (isem0, isem1), (ssem0, ssem1)
        c = lax.axis_index("c")
        s = lax.axis_index("s")
        lo = c * HALF
        base = s * e_per_tile

        def load_idx(cj, b):
            eb = pl.multiple_of(base + lax.min(cj, n_chunks - 1) * CH, CH)
            pltpu.async_copy(dst_hbm.at[pl.ds(eb, CH)], dstv[b], isem[b])
            if gather:
                pltpu.async_copy(src_hbm.at[pl.ds(eb, CH)], srcv[b], isem[b])

        def wait_idx(b):
            pltpu.make_async_copy(dst_hbm.at[pl.ds(0, CH)], dstv[b],
                                  isem[b]).wait()
            if gather:
                pltpu.make_async_copy(src_hbm.at[pl.ds(0, CH)], srcv[b],
                                      isem[b]).wait()

        def compute_idx(b):
            for k in range(CH16):
                sl = pl.ds(k * 16, 16)
                d16 = dstv[b][sl]
                l16 = d16 - lo
                ok = (l16 >= 0) & (l16 < HALF)
                d2 = 2 * l16
                dstA[b][sl] = jnp.where(ok, d2, SENT)
                if gather:
                    dstB[b][sl] = jnp.where(ok, d2 + 1, SENT)
                    s16 = srcv[b][sl]
                    sph = jnp.where(s16 >= HALF, s16 + (HALFP - HALF), s16)
                    srcA[b][sl] = 2 * sph
                    srcB[b][sl] = 2 * sph + 1

        def fire_gather(b):
            pltpu.async_copy(g_hbm.at[srcA[b]], rowsA[b], gsem[b])
            pltpu.async_copy(g_hbm.at[srcB[b]], rowsB[b], gsem[b])

        def wait_gather(b):
            pltpu.make_async_copy(g_hbm.at[pl.ds(0, CH)], rowsA[b],
                                  gsem[b]).wait()
            pltpu.make_async_copy(g_hbm.at[pl.ds(0, CH)], rowsB[b],
                                  gsem[b]).wait()

        def fire_scatter(b):
            pltpu.async_copy(rowsA[b],
                             acc.at[plsc.Indices(dstA[b], ignored_value=SENT)],
                             ssem[b], add=True)
            if gather:
                pltpu.async_copy(
                    rowsB[b],
                    acc.at[plsc.Indices(dstB[b], ignored_value=SENT)],
                    ssem[b], add=True)

        def wait_scatter(b):
            pltpu.make_async_copy(rowsA[b], acc.at[pl.ds(0, CH)],
                                  ssem[b]).wait()
            if gather:
                pltpu.make_async_copy(rowsB[b], acc.at[pl.ds(0, CH)],
                                      ssem[b]).wait()

        if gather:
            pltpu.sync_copy(g_hbm.at[pl.ds(c * (2 * HALFP) + s * RPT, RPT)],
                            acc.at[pl.ds(s * RPT, RPT)])
        else:
            pltpu.sync_copy(ones_hbm, acc.at[pl.ds(s * RPT, RPT)])
            pltpu.sync_copy(ones_hbm.at[pl.ds(0, CH)], rowsA[0])
            pltpu.sync_copy(ones_hbm.at[pl.ds(0, CH)], rowsB[0])
            pltpu.sync_copy(ones_hbm.at[pl.ds(0, CH)], rowsA[1])
            pltpu.sync_copy(ones_hbm.at[pl.ds(0, CH)], rowsB[1])
        plsc.subcore_barrier()

        if gather:
            load_idx(0, 0)
            load_idx(1, 1)
            wait_idx(0)
            compute_idx(0)
            fire_gather(0)
            wait_idx(1)
            compute_idx(1)
            wait_gather(0)
            fire_scatter(0)
            fire_gather(1)
            load_idx(2, 0)
            wait_idx(0)
            wait_scatter(0)
            compute_idx(0)
            wait_gather(1)
            fire_scatter(1)
            fire_gather(0)
            load_idx(3, 1)

            def pair(g, carry):
                wait_idx(1)
                wait_scatter(1)
                compute_idx(1)
                wait_gather(0)
                fire_scatter(0)
                fire_gather(1)
                load_idx(2 * g + 2, 0)
                wait_idx(0)
                wait_scatter(0)
                compute_idx(0)
                wait_gather(1)
                fire_scatter(1)
                fire_gather(0)
                load_idx(2 * g + 3, 1)
                return carry

            lax.fori_loop(1, n_pairs, pair, 0)
            wait_idx(1)
            wait_scatter(1)
            wait_gather(0)
        else:
            load_idx(0, 0)
            load_idx(1, 1)
            wait_idx(0)
            compute_idx(0)
            fire_scatter(0)
            load_idx(2, 0)
            wait_idx(1)
            compute_idx(1)
            fire_scatter(1)
            load_idx(3, 1)

            def pair(g, carry):
                wait_idx(0)
                wait_scatter(0)
                compute_idx(0)
                fire_scatter(0)
                load_idx(2 * g + 2, 0)
                wait_idx(1)
                wait_scatter(1)
                compute_idx(1)
                fire_scatter(1)
                load_idx(2 * g + 3, 1)
                return carry

            lax.fori_loop(1, n_pairs, pair, 0)
            wait_idx(0)
            wait_idx(1)
            wait_scatter(0)
            wait_scatter(1)
        plsc.subcore_barrier()
        pltpu.sync_copy(acc.at[pl.ds(s * RPT, RPT)],
                        out_hbm.at[pl.ds(c * (2 * HALFP) + s * RPT, RPT)])

    return prop


def _mm_first():
    def body(x_ref, dis_ref, w_ref, out_ref):
        o = jnp.dot(x_ref[...], w_ref[...],
                    preferred_element_type=jnp.float32)
        out_ref[...] = o * dis_ref[...]

    return pl.pallas_call(
        body,
        grid=(NROWS // _BR,),
        in_specs=[
            pl.BlockSpec((_BR, D), lambda i: (i, 0)),
            pl.BlockSpec((_BR, D), lambda i: (i, 0)),
            pl.BlockSpec((D, D), lambda i: (0, 0)),
        ],
        out_specs=pl.BlockSpec((_BR, D), lambda i: (i, 0)),
        out_shape=jax.ShapeDtypeStruct((NROWS, D), jnp.float32),
    )


def _mm_mid(relu):
    def body(s_ref, dis_ref, b_ref, w_ref, out_ref):
        t = s_ref[...] * dis_ref[...] + b_ref[...]
        if relu:
            t = jnp.maximum(t, 0.0)
        o = jnp.dot(t, w_ref[...], preferred_element_type=jnp.float32)
        out_ref[...] = o * dis_ref[...]

    return pl.pallas_call(
        body,
        grid=(NROWS // _BR,),
        in_specs=[
            pl.BlockSpec((_BR, D), lambda i: (i, 0)),
            pl.BlockSpec((_BR, D), lambda i: (i, 0)),
            pl.BlockSpec((1, D), lambda i: (0, 0)),
            pl.BlockSpec((D, D), lambda i: (0, 0)),
        ],
        out_specs=pl.BlockSpec((_BR, D), lambda i: (i, 0)),
        out_shape=jax.ShapeDtypeStruct((NROWS, D), jnp.float32),
    )


def _rsqrt_body(deg_ref, out_ref):
    d0 = deg_ref[:, 0:1]
    out_ref[...] = jnp.broadcast_to(lax.rsqrt(d0), (_BR, D))


_rsqrt_kernel = pl.pallas_call(
    _rsqrt_body,
    grid=(NROWS // _BR,),
    in_specs=[pl.BlockSpec((_BR, D), lambda i: (i, 0))],
    out_specs=pl.BlockSpec((_BR, D), lambda i: (i, 0)),
    out_shape=jax.ShapeDtypeStruct((NROWS, D), jnp.float32),
)


def _finish_body(s_ref, dis_ref, b_ref, out_ref):
    out_ref[...] = s_ref[...] * dis_ref[...] + b_ref[...]


_finish = pl.pallas_call(
    _finish_body,
    grid=(NROWS // _BR,),
    in_specs=[
        pl.BlockSpec((_BR, D), lambda i: (i, 0)),
        pl.BlockSpec((_BR, D), lambda i: (i, 0)),
        pl.BlockSpec((1, D), lambda i: (0, 0)),
    ],
    out_specs=pl.BlockSpec((_BR, D), lambda i: (i, 0)),
    out_shape=jax.ShapeDtypeStruct((NROWS, D), jnp.float32),
)


def kernel(data, x, edge_index, W_e1, b_e1, W_e2, b_e2, W_d1, b_d1, W_d2, b_d2):
    src = edge_index[0]
    dst = edge_index[1]
    e = src.shape[0]
    epad = -(-e // (16 * CH * 2)) * (16 * CH * 2)
    if epad != e:
        pad = jnp.arange(epad - e, dtype=jnp.int32)
        src_p = jnp.concatenate([src, (pad * 2003) % N])
        dst_p = jnp.concatenate([dst, jnp.full((epad - e,), N, jnp.int32)])
    else:
        src_p, dst_p = src, dst

    prop_deg = _make_prop(epad, gather=False)
    prop = _make_prop(epad, gather=True)

    def run_prop(g):
        out2 = prop(g.reshape(2 * NROWS, HH), src_p, dst_p)
        return out2.reshape(NROWS, D)

    ones_blk = jnp.ones((RPT, HH), jnp.float32)
    deg2 = prop_deg(ones_blk, dst_p)
    dis = _rsqrt_kernel(deg2.reshape(NROWS, D))

    zpad = jnp.zeros((HALFP - HALF, D), jnp.float32)
    xp = jnp.concatenate([x[:HALF], zpad, x[HALF:], zpad], axis=0)
    g1 = _mm_first()(xp, dis, W_e1)
    s1 = run_prop(g1)
    g2 = _mm_mid(True)(s1, dis, b_e1.reshape(1, D), W_e2)
    s2 = run_prop(g2)
    g3 = _mm_mid(False)(s2, dis, b_e2.reshape(1, D), W_d1)
    s3 = run_prop(g3)
    g4 = _mm_mid(True)(s3, dis, b_d1.reshape(1, D), W_d2)
    s4 = run_prop(g4)
    y = _finish(s4, dis, b_d2.reshape(1, D))

    x_ = jnp.concatenate([y[:HALF], y[HALFP:HALFP + HALF]], axis=0)
    return (x_, 1, 1)


# speedup vs baseline: 8.8883x; 1.0977x over previous
import functools

import jax
import jax.numpy as jnp
from jax import lax
from jax.experimental import pallas as pl
from jax.experimental.pallas import tpu as pltpu
from jax.experimental.pallas import tpu_sc as plsc

N = 10000
D = 256
HALF = 5000
HALFP = 5120
NROWS = 2 * HALFP
HH = 128
SENT = -1
ACC_ROWS = 2 * HALFP
CH = 80
RPT = 2 * HALFP // 16
CH16 = CH // 16
_BR = 1024


def _make_prop(epad, gather):
    e_per_tile = epad // 16
    n_chunks = e_per_tile // CH
    n_pairs = n_chunks // 2
    mesh = plsc.VectorSubcoreMesh(core_axis_name="c", subcore_axis_name="s")

    def _vec2(shape, dt):
        return [pltpu.VMEM(shape, dt), pltpu.VMEM(shape, dt)]

    scratch = (
        _vec2((CH,), jnp.int32)
        + _vec2((CH,), jnp.int32)
        + _vec2((CH,), jnp.int32)
        + _vec2((CH, HH), jnp.float32)
        + _vec2((CH, HH), jnp.float32)
        + [pltpu.VMEM_SHARED((ACC_ROWS, HH), jnp.float32)]
        + [pltpu.SemaphoreType.DMA] * 4
    )
    if gather:
        scratch = (
            _vec2((CH,), jnp.int32)
            + _vec2((CH,), jnp.int32)
            + _vec2((CH,), jnp.int32)
            + scratch
            + [pltpu.SemaphoreType.DMA] * 2
        )

    @functools.partial(pl.kernel, mesh=mesh,
                       out_type=jax.ShapeDtypeStruct((2 * NROWS, HH),
                                                     jnp.float32),
                       scratch_types=scratch)
    def prop(*refs):
        if gather:
            (g_hbm, src_hbm, dst_hbm, out_hbm,
             srcv0, srcv1, srcA0, srcA1, srcB0, srcB1,
             dstv0, dstv1, dstA0, dstA1, dstB0, dstB1,
             rowsA0, rowsA1, rowsB0, rowsB1, acc,
             isem0, isem1, ssem0, ssem1, gsem0, gsem1) = refs
            srcv, srcA, srcB = (srcv0, srcv1), (srcA0, srcA1), (srcB0, srcB1)
            gsem = (gsem0, gsem1)
        else:
            (ones_hbm, dst_hbm, out_hbm,
             dstv0, dstv1, dstA0, dstA1, dstB0, dstB1,
             rowsA0, rowsA1, rowsB0, rowsB1, acc,
             isem0, isem1, ssem0, ssem1) = refs
        dstv, dstA, dstB = (dstv0, dstv1), (dstA0, dstA1), (dstB0, dstB1)
        rowsA, rowsB = (rowsA0, rowsA1), (rowsB0, rowsB1)
        isem, ssem = (isem0, isem1), (ssem0, ssem1)
        c = lax.axis_index("c")
        s = lax.axis_index("s")
        lo = c * HALF
        base = s * e_per_tile

        def load_idx(cj, b):
            eb = pl.multiple_of(base + lax.min(cj, n_chunks - 1) * CH, CH)
            pltpu.async_copy(dst_hbm.at[pl.ds(eb, CH)], dstv[b], isem[b])
            if gather:
                pltpu.async_copy(src_hbm.at[pl.ds(eb, CH)], srcv[b], isem[b])

        def wait_idx(b):
            pltpu.make_async_copy(dst_hbm.at[pl.ds(0, CH)], dstv[b],
                                  isem[b]).wait()
            if gather:
                pltpu.make_async_copy(src_hbm.at[pl.ds(0, CH)], srcv[b],
                                      isem[b]).wait()

        def compute_idx(b):
            for k in range(CH16):
                sl = pl.ds(k * 16, 16)
                d16 = dstv[b][sl]
                l16 = d16 - lo
                ok = (l16 >= 0) & (l16 < HALF)
                d2 = 2 * l16
                dstA[b][sl] = jnp.where(ok, d2, SENT)
                if gather:
                    dstB[b][sl] = jnp.where(ok, d2 + 1, SENT)
                    s16 = srcv[b][sl]
                    sph = jnp.where(s16 >= HALF, s16 + (HALFP - HALF), s16)
                    s2 = 2 * sph
                    srcA[b][sl] = jnp.where(ok, s2, SENT)
                    srcB[b][sl] = jnp.where(ok, s2 + 1, SENT)

        def fire_gather(b):
            pltpu.async_copy(g_hbm.at[plsc.Indices(srcA[b],
                                                   ignored_value=SENT)],
                             rowsA[b], gsem[b])
            pltpu.async_copy(g_hbm.at[plsc.Indices(srcB[b],
                                                   ignored_value=SENT)],
                             rowsB[b], gsem[b])

        def wait_gather(b):
            pltpu.make_async_copy(g_hbm.at[pl.ds(0, CH)], rowsA[b],
                                  gsem[b]).wait()
            pltpu.make_async_copy(g_hbm.at[pl.ds(0, CH)], rowsB[b],
                                  gsem[b]).wait()

        def fire_scatter(b):
            pltpu.async_copy(rowsA[b],
                             acc.at[plsc.Indices(dstA[b], ignored_value=SENT)],
                             ssem[b], add=True)
            if gather:
                pltpu.async_copy(
                    rowsB[b],
                    acc.at[plsc.Indices(dstB[b], ignored_value=SENT)],
                    ssem[b], add=True)

        def wait_scatter(b):
            pltpu.make_async_copy(rowsA[b], acc.at[pl.ds(0, CH)],
                                  ssem[b]).wait()
            if gather:
                pltpu.make_async_copy(rowsB[b], acc.at[pl.ds(0, CH)],
                                      ssem[b]).wait()

        if gather:
            pltpu.sync_copy(g_hbm.at[pl.ds(c * (2 * HALFP) + s * RPT, RPT)],
                            acc.at[pl.ds(s * RPT, RPT)])
        else:
            pltpu.sync_copy(ones_hbm, acc.at[pl.ds(s * RPT, RPT)])
            pltpu.sync_copy(ones_hbm.at[pl.ds(0, CH)], rowsA[0])
            pltpu.sync_copy(ones_hbm.at[pl.ds(0, CH)], rowsB[0])
            pltpu.sync_copy(ones_hbm.at[pl.ds(0, CH)], rowsA[1])
            pltpu.sync_copy(ones_hbm.at[pl.ds(0, CH)], rowsB[1])
        plsc.subcore_barrier()

        if gather:
            load_idx(0, 0)
            load_idx(1, 1)
            wait_idx(0)
            compute_idx(0)
            fire_gather(0)
            wait_idx(1)
            compute_idx(1)
            wait_gather(0)
            fire_scatter(0)
            fire_gather(1)
            load_idx(2, 0)
            wait_idx(0)
            wait_scatter(0)
            compute_idx(0)
            wait_gather(1)
            fire_scatter(1)
            fire_gather(0)
            load_idx(3, 1)

            def pair(g, carry):
                wait_idx(1)
                wait_scatter(1)
                compute_idx(1)
                wait_gather(0)
                fire_scatter(0)
                fire_gather(1)
                load_idx(2 * g + 2, 0)
                wait_idx(0)
                wait_scatter(0)
                compute_idx(0)
                wait_gather(1)
                fire_scatter(1)
                fire_gather(0)
                load_idx(2 * g + 3, 1)
                return carry

            lax.fori_loop(1, n_pairs, pair, 0)
            wait_idx(1)
            wait_scatter(1)
            wait_gather(0)
        else:
            load_idx(0, 0)
            load_idx(1, 1)
            wait_idx(0)
            compute_idx(0)
            fire_scatter(0)
            load_idx(2, 0)
            wait_idx(1)
            compute_idx(1)
            fire_scatter(1)
            load_idx(3, 1)

            def pair(g, carry):
                wait_idx(0)
                wait_scatter(0)
                compute_idx(0)
                fire_scatter(0)
                load_idx(2 * g + 2, 0)
                wait_idx(1)
                wait_scatter(1)
                compute_idx(1)
                fire_scatter(1)
                load_idx(2 * g + 3, 1)
                return carry

            lax.fori_loop(1, n_pairs, pair, 0)
            wait_idx(0)
            wait_idx(1)
            wait_scatter(0)
            wait_scatter(1)
        plsc.subcore_barrier()
        pltpu.sync_copy(acc.at[pl.ds(s * RPT, RPT)],
                        out_hbm.at[pl.ds(c * (2 * HALFP) + s * RPT, RPT)])

    return prop


def _mm_first():
    def body(x_ref, dis_ref, w_ref, out_ref):
        o = jnp.dot(x_ref[...], w_ref[...],
                    preferred_element_type=jnp.float32)
        out_ref[...] = o * dis_ref[...]

    return pl.pallas_call(
        body,
        grid=(NROWS // _BR,),
        in_specs=[
            pl.BlockSpec((_BR, D), lambda i: (i, 0)),
            pl.BlockSpec((_BR, D), lambda i: (i, 0)),
            pl.BlockSpec((D, D), lambda i: (0, 0)),
        ],
        out_specs=pl.BlockSpec((_BR, D), lambda i: (i, 0)),
        out_shape=jax.ShapeDtypeStruct((NROWS, D), jnp.float32),
    )


def _mm_mid(relu):
    def body(s_ref, dis_ref, b_ref, w_ref, out_ref):
        t = s_ref[...] * dis_ref[...] + b_ref[...]
        if relu:
            t = jnp.maximum(t, 0.0)
        o = jnp.dot(t, w_ref[...], preferred_element_type=jnp.float32)
        out_ref[...] = o * dis_ref[...]

    return pl.pallas_call(
        body,
        grid=(NROWS // _BR,),
        in_specs=[
            pl.BlockSpec((_BR, D), lambda i: (i, 0)),
            pl.BlockSpec((_BR, D), lambda i: (i, 0)),
            pl.BlockSpec((1, D), lambda i: (0, 0)),
            pl.BlockSpec((D, D), lambda i: (0, 0)),
        ],
        out_specs=pl.BlockSpec((_BR, D), lambda i: (i, 0)),
        out_shape=jax.ShapeDtypeStruct((NROWS, D), jnp.float32),
    )


def _rsqrt_body(deg_ref, out_ref):
    d0 = deg_ref[:, 0:1]
    out_ref[...] = jnp.broadcast_to(lax.rsqrt(d0), (_BR, D))


_rsqrt_kernel = pl.pallas_call(
    _rsqrt_body,
    grid=(NROWS // _BR,),
    in_specs=[pl.BlockSpec((_BR, D), lambda i: (i, 0))],
    out_specs=pl.BlockSpec((_BR, D), lambda i: (i, 0)),
    out_shape=jax.ShapeDtypeStruct((NROWS, D), jnp.float32),
)


def _finish_body(s_ref, dis_ref, b_ref, out_ref):
    out_ref[...] = s_ref[...] * dis_ref[...] + b_ref[...]


_finish = pl.pallas_call(
    _finish_body,
    grid=(NROWS // _BR,),
    in_specs=[
        pl.BlockSpec((_BR, D), lambda i: (i, 0)),
        pl.BlockSpec((_BR, D), lambda i: (i, 0)),
        pl.BlockSpec((1, D), lambda i: (0, 0)),
    ],
    out_specs=pl.BlockSpec((_BR, D), lambda i: (i, 0)),
    out_shape=jax.ShapeDtypeStruct((NROWS, D), jnp.float32),
)


def kernel(data, x, edge_index, W_e1, b_e1, W_e2, b_e2, W_d1, b_d1, W_d2, b_d2):
    src = edge_index[0]
    dst = edge_index[1]
    e = src.shape[0]
    epad = -(-e // (16 * CH * 2)) * (16 * CH * 2)
    if epad != e:
        pad = jnp.arange(epad - e, dtype=jnp.int32)
        src_p = jnp.concatenate([src, (pad * 2003) % N])
        dst_p = jnp.concatenate([dst, jnp.full((epad - e,), N, jnp.int32)])
    else:
        src_p, dst_p = src, dst

    prop_deg = _make_prop(epad, gather=False)
    prop = _make_prop(epad, gather=True)

    def run_prop(g):
        out2 = prop(g.reshape(2 * NROWS, HH), src_p, dst_p)
        return out2.reshape(NROWS, D)

    ones_blk = jnp.ones((RPT, HH), jnp.float32)
    deg2 = prop_deg(ones_blk, dst_p)
    dis = _rsqrt_kernel(deg2.reshape(NROWS, D))

    zpad = jnp.zeros((HALFP - HALF, D), jnp.float32)
    xp = jnp.concatenate([x[:HALF], zpad, x[HALF:], zpad], axis=0)
    g1 = _mm_first()(xp, dis, W_e1)
    s1 = run_prop(g1)
    g2 = _mm_mid(True)(s1, dis, b_e1.reshape(1, D), W_e2)
    s2 = run_prop(g2)
    g3 = _mm_mid(False)(s2, dis, b_e2.reshape(1, D), W_d1)
    s3 = run_prop(g3)
    g4 = _mm_mid(True)(s3, dis, b_d1.reshape(1, D), W_d2)
    s4 = run_prop(g4)
    y = _finish(s4, dis, b_d2.reshape(1, D))

    x_ = jnp.concatenate([y[:HALF], y[HALFP:HALFP + HALF]], axis=0)
    return (x_, 1, 1)


# speedup vs baseline: 11.1561x; 1.2551x over previous
import functools

import jax
import jax.numpy as jnp
from jax import lax
from jax.experimental import pallas as pl
from jax.experimental.pallas import tpu as pltpu
from jax.experimental.pallas import tpu_sc as plsc

N = 10000
D = 256
HALF = 5000
HALFP = 5120
NROWS = 2 * HALFP
HH = 128
SENT = -1
ACC_ROWS = 10112
CH = 48
NB = 3
RPT = 632
CH16 = CH // 16
_BR = 1024


def _make_prop(epad, gather):
    e_per_tile = epad // 16
    n_chunks = e_per_tile // CH
    n_groups = n_chunks // NB
    mesh = plsc.VectorSubcoreMesh(core_axis_name="c", subcore_axis_name="s")

    def _vecs(shape, dt):
        return [pltpu.VMEM(shape, dt) for _ in range(NB)]

    scratch = (
        _vecs((CH,), jnp.int32)
        + _vecs((CH,), jnp.int32)
        + _vecs((CH,), jnp.int32)
        + _vecs((CH, HH), jnp.float32)
        + _vecs((CH, HH), jnp.float32)
        + [pltpu.VMEM_SHARED((ACC_ROWS, HH), jnp.float32)]
        + [pltpu.SemaphoreType.DMA] * (2 * NB)
    )
    if gather:
        scratch = (
            _vecs((CH,), jnp.int32)
            + _vecs((CH,), jnp.int32)
            + _vecs((CH,), jnp.int32)
            + scratch
            + [pltpu.SemaphoreType.DMA] * NB
        )

    @functools.partial(pl.kernel, mesh=mesh,
                       out_type=jax.ShapeDtypeStruct((2 * NROWS, HH),
                                                     jnp.float32),
                       scratch_types=scratch)
    def prop(*refs):
        if gather:
            g_hbm, src_hbm, dst_hbm, out_hbm = refs[:4]
            r = list(refs[4:])
            srcv, srcA, srcB = r[0:NB], r[NB:2*NB], r[2*NB:3*NB]
            r = r[3*NB:]
        else:
            ones_hbm, dst_hbm, out_hbm = refs[:3]
            r = list(refs[3:])
        dstv, dstA, dstB = r[0:NB], r[NB:2*NB], r[2*NB:3*NB]
        rowsA, rowsB = r[3*NB:4*NB], r[4*NB:5*NB]
        acc = r[5*NB]
        isem, ssem = r[5*NB+1:5*NB+1+NB], r[5*NB+1+NB:5*NB+1+2*NB]
        if gather:
            gsem = r[5*NB+1+2*NB:5*NB+1+3*NB]
        c = lax.axis_index("c")
        s = lax.axis_index("s")
        lo = c * HALF
        base = s * e_per_tile

        def load_idx(cj, b):
            eb = pl.multiple_of(base + lax.min(cj, n_chunks - 1) * CH, CH)
            pltpu.async_copy(dst_hbm.at[pl.ds(eb, CH)], dstv[b], isem[b])
            if gather:
                pltpu.async_copy(src_hbm.at[pl.ds(eb, CH)], srcv[b], isem[b])

        def wait_idx(b):
            pltpu.make_async_copy(dst_hbm.at[pl.ds(0, CH)], dstv[b],
                                  isem[b]).wait()
            if gather:
                pltpu.make_async_copy(src_hbm.at[pl.ds(0, CH)], srcv[b],
                                      isem[b]).wait()

        def compute_idx(b):
            for k in range(CH16):
                sl = pl.ds(k * 16, 16)
                d16 = dstv[b][sl]
                l16 = d16 - lo
                ok = (l16 >= 0) & (l16 < HALF)
                d2 = 2 * l16
                dstA[b][sl] = jnp.where(ok, d2, SENT)
                if gather:
                    dstB[b][sl] = jnp.where(ok, d2 + 1, SENT)
                    s16 = srcv[b][sl]
                    sph = jnp.where(s16 >= HALF, s16 + (HALFP - HALF), s16)
                    s2 = 2 * sph
                    srcA[b][sl] = jnp.where(ok, s2, SENT)
                    srcB[b][sl] = jnp.where(ok, s2 + 1, SENT)

        def fire_gather(b):
            pltpu.async_copy(g_hbm.at[plsc.Indices(srcA[b],
                                                   ignored_value=SENT)],
                             rowsA[b], gsem[b])
            pltpu.async_copy(g_hbm.at[plsc.Indices(srcB[b],
                                                   ignored_value=SENT)],
                             rowsB[b], gsem[b])

        def wait_gather(b):
            pltpu.make_async_copy(g_hbm.at[pl.ds(0, CH)], rowsA[b],
                                  gsem[b]).wait()
            pltpu.make_async_copy(g_hbm.at[pl.ds(0, CH)], rowsB[b],
                                  gsem[b]).wait()

        def fire_scatter(b):
            pltpu.async_copy(rowsA[b],
                             acc.at[plsc.Indices(dstA[b], ignored_value=SENT)],
                             ssem[b], add=True)
            if gather:
                pltpu.async_copy(
                    rowsB[b],
                    acc.at[plsc.Indices(dstB[b], ignored_value=SENT)],
                    ssem[b], add=True)

        def wait_scatter(b):
            pltpu.make_async_copy(rowsA[b], acc.at[pl.ds(0, CH)],
                                  ssem[b]).wait()
            if gather:
                pltpu.make_async_copy(rowsB[b], acc.at[pl.ds(0, CH)],
                                      ssem[b]).wait()

        if gather:
            pltpu.sync_copy(g_hbm.at[pl.ds(c * (2 * HALFP) + s * RPT, RPT)],
                            acc.at[pl.ds(s * RPT, RPT)])
        else:
            pltpu.sync_copy(ones_hbm, acc.at[pl.ds(s * RPT, RPT)])
            for b in range(NB):
                pltpu.sync_copy(ones_hbm.at[pl.ds(0, CH)], rowsA[b])
        plsc.subcore_barrier()

        if gather:
            for b in range(NB):
                load_idx(b, b)
            for j in range(NB):
                b = j % NB
                wait_idx(b)
                compute_idx(b)
                fire_gather(b)
                load_idx(j + NB, b)
                if j > 0:
                    pb = (b + NB - 1) % NB
                    wait_gather(pb)
                    fire_scatter(pb)

            def group(g, carry):
                for b in range(NB):
                    wait_idx(b)
                    wait_scatter(b)
                    compute_idx(b)
                    fire_gather(b)
                    load_idx(g * NB + b + NB, b)
                    pb = (b + NB - 1) % NB
                    wait_gather(pb)
                    fire_scatter(pb)
                return carry

            lax.fori_loop(1, n_groups, group, 0)
            lastb = (n_chunks - 1) % NB
            wait_gather(lastb)
            fire_scatter(lastb)
            for b in range(NB):
                wait_scatter(b)
                pltpu.make_async_copy(dst_hbm.at[pl.ds(0, CH)], dstv[b],
                                      isem[b]).wait()
                pltpu.make_async_copy(src_hbm.at[pl.ds(0, CH)], srcv[b],
                                      isem[b]).wait()
        else:
            for b in range(NB):
                load_idx(b, b)
            for j in range(NB):
                b = j % NB
                wait_idx(b)
                compute_idx(b)
                fire_scatter(b)
                load_idx(j + NB, b)

            def group(g, carry):
                for b in range(NB):
                    wait_idx(b)
                    wait_scatter(b)
                    compute_idx(b)
                    fire_scatter(b)
                    load_idx(g * NB + b + NB, b)
                return carry

            lax.fori_loop(1, n_groups, group, 0)
            for b in range(NB):
                wait_scatter(b)
                pltpu.make_async_copy(dst_hbm.at[pl.ds(0, CH)], dstv[b],
                                      isem[b]).wait()
        plsc.subcore_barrier()
        pltpu.sync_copy(acc.at[pl.ds(s * RPT, RPT)],
                        out_hbm.at[pl.ds(c * (2 * HALFP) + s * RPT, RPT)])

    return prop


def _mm_first():
    def body(x_ref, dis_ref, w_ref, out_ref):
        o = jnp.dot(x_ref[...], w_ref[...],
                    preferred_element_type=jnp.float32)
        out_ref[...] = o * dis_ref[...]

    return pl.pallas_call(
        body,
        grid=(NROWS // _BR,),
        in_specs=[
            pl.BlockSpec((_BR, D), lambda i: (i, 0)),
            pl.BlockSpec((_BR, D), lambda i: (i, 0)),
            pl.BlockSpec((D, D), lambda i: (0, 0)),
        ],
        out_specs=pl.BlockSpec((_BR, D), lambda i: (i, 0)),
        out_shape=jax.ShapeDtypeStruct((NROWS, D), jnp.float32),
    )


def _mm_mid(relu):
    def body(s_ref, dis_ref, b_ref, w_ref, out_ref):
        t = s_ref[...] * dis_ref[...] + b_ref[...]
        if relu:
            t = jnp.maximum(t, 0.0)
        o = jnp.dot(t, w_ref[...], preferred_element_type=jnp.float32)
        out_ref[...] = o * dis_ref[...]

    return pl.pallas_call(
        body,
        grid=(NROWS // _BR,),
        in_specs=[
            pl.BlockSpec((_BR, D), lambda i: (i, 0)),
            pl.BlockSpec((_BR, D), lambda i: (i, 0)),
            pl.BlockSpec((1, D), lambda i: (0, 0)),
            pl.BlockSpec((D, D), lambda i: (0, 0)),
        ],
        out_specs=pl.BlockSpec((_BR, D), lambda i: (i, 0)),
        out_shape=jax.ShapeDtypeStruct((NROWS, D), jnp.float32),
    )


def _rsqrt_body(deg_ref, out_ref):
    d0 = deg_ref[:, 0:1]
    out_ref[...] = jnp.broadcast_to(lax.rsqrt(d0), (_BR, D))


_rsqrt_kernel = pl.pallas_call(
    _rsqrt_body,
    grid=(NROWS // _BR,),
    in_specs=[pl.BlockSpec((_BR, D), lambda i: (i, 0))],
    out_specs=pl.BlockSpec((_BR, D), lambda i: (i, 0)),
    out_shape=jax.ShapeDtypeStruct((NROWS, D), jnp.float32),
)


def _finish_body(s_ref, dis_ref, b_ref, out_ref):
    out_ref[...] = s_ref[...] * dis_ref[...] + b_ref[...]


_finish = pl.pallas_call(
    _finish_body,
    grid=(NROWS // _BR,),
    in_specs=[
        pl.BlockSpec((_BR, D), lambda i: (i, 0)),
        pl.BlockSpec((_BR, D), lambda i: (i, 0)),
        pl.BlockSpec((1, D), lambda i: (0, 0)),
    ],
    out_specs=pl.BlockSpec((_BR, D), lambda i: (i, 0)),
    out_shape=jax.ShapeDtypeStruct((NROWS, D), jnp.float32),
)


def kernel(data, x, edge_index, W_e1, b_e1, W_e2, b_e2, W_d1, b_d1, W_d2, b_d2):
    src = edge_index[0]
    dst = edge_index[1]
    e = src.shape[0]
    epad = -(-e // (16 * CH * NB)) * (16 * CH * NB)
    if epad != e:
        pad = jnp.arange(epad - e, dtype=jnp.int32)
        src_p = jnp.concatenate([src, (pad * 2003) % N])
        dst_p = jnp.concatenate([dst, jnp.full((epad - e,), N, jnp.int32)])
    else:
        src_p, dst_p = src, dst

    prop_deg = _make_prop(epad, gather=False)
    prop = _make_prop(epad, gather=True)

    def run_prop(g):
        out2 = prop(g.reshape(2 * NROWS, HH), src_p, dst_p)
        return out2.reshape(NROWS, D)

    ones_blk = jnp.ones((RPT, HH), jnp.float32)
    deg2 = prop_deg(ones_blk, dst_p)
    dis = _rsqrt_kernel(deg2.reshape(NROWS, D))

    zpad = jnp.zeros((HALFP - HALF, D), jnp.float32)
    xp = jnp.concatenate([x[:HALF], zpad, x[HALF:], zpad], axis=0)
    g1 = _mm_first()(xp, dis, W_e1)
    s1 = run_prop(g1)
    g2 = _mm_mid(True)(s1, dis, b_e1.reshape(1, D), W_e2)
    s2 = run_prop(g2)
    g3 = _mm_mid(False)(s2, dis, b_e2.reshape(1, D), W_d1)
    s3 = run_prop(g3)
    g4 = _mm_mid(True)(s3, dis, b_d1.reshape(1, D), W_d2)
    s4 = run_prop(g4)
    y = _finish(s4, dis, b_d2.reshape(1, D))

    x_ = jnp.concatenate([y[:HALF], y[HALFP:HALFP + HALF]], axis=0)
    return (x_, 1, 1)


# speedup vs baseline: 11.2774x; 1.0109x over previous
import functools

import jax
import jax.numpy as jnp
from jax import lax
from jax.experimental import pallas as pl
from jax.experimental.pallas import tpu as pltpu
from jax.experimental.pallas import tpu_sc as plsc

N = 10000
D = 256
HALF = 5000
HALFP = 5120
NROWS = 2 * HALFP
HH = 128
SENT = -1
ACC_ROWS = 10112
CH = 48
NB = 3
RPT = 632
CH16 = CH // 16
_BR = 1024


def _make_prop(epad, gather):
    e_per_tile = epad // 16
    n_chunks = e_per_tile // CH
    n_groups = n_chunks // NB
    mesh = plsc.VectorSubcoreMesh(core_axis_name="c", subcore_axis_name="s")

    def _vecs(shape, dt):
        return [pltpu.VMEM(shape, dt) for _ in range(NB)]

    scratch = (
        _vecs((CH,), jnp.int32)
        + _vecs((CH,), jnp.int32)
        + _vecs((CH,), jnp.int32)
        + _vecs((CH, HH), jnp.float32)
        + _vecs((CH, HH), jnp.float32)
        + [pltpu.VMEM_SHARED((ACC_ROWS, HH), jnp.float32)]
        + [pltpu.SemaphoreType.DMA] * (2 * NB)
    )
    if gather:
        scratch = (
            _vecs((CH,), jnp.int32)
            + _vecs((CH,), jnp.int32)
            + _vecs((CH,), jnp.int32)
            + scratch
            + [pltpu.SemaphoreType.DMA] * NB
        )

    @functools.partial(pl.kernel, mesh=mesh,
                       out_type=jax.ShapeDtypeStruct((2 * NROWS, HH),
                                                     jnp.float32),
                       scratch_types=scratch)
    def prop(*refs):
        if gather:
            g_hbm, src_hbm, dst_hbm, out_hbm = refs[:4]
            r = list(refs[4:])
            srcv, srcA, srcB = r[0:NB], r[NB:2*NB], r[2*NB:3*NB]
            r = r[3*NB:]
        else:
            ones_hbm, dst_hbm, out_hbm = refs[:3]
            r = list(refs[3:])
        dstv, dstA, dstB = r[0:NB], r[NB:2*NB], r[2*NB:3*NB]
        rowsA, rowsB = r[3*NB:4*NB], r[4*NB:5*NB]
        acc = r[5*NB]
        isem, ssem = r[5*NB+1:5*NB+1+NB], r[5*NB+1+NB:5*NB+1+2*NB]
        if gather:
            gsem = r[5*NB+1+2*NB:5*NB+1+3*NB]
        c = lax.axis_index("c")
        s = lax.axis_index("s")
        lo = c * HALF
        base = s * e_per_tile

        def load_idx(cj, b):
            eb = pl.multiple_of(base + lax.min(cj, n_chunks - 1) * CH, CH)
            pltpu.async_copy(dst_hbm.at[pl.ds(eb, CH)], dstv[b], isem[b])
            if gather:
                pltpu.async_copy(src_hbm.at[pl.ds(eb, CH)], srcv[b], isem[b])

        def wait_idx(b):
            pltpu.make_async_copy(dst_hbm.at[pl.ds(0, CH)], dstv[b],
                                  isem[b]).wait()
            if gather:
                pltpu.make_async_copy(src_hbm.at[pl.ds(0, CH)], srcv[b],
                                      isem[b]).wait()

        def compute_idx(b):
            for k in range(CH16):
                sl = pl.ds(k * 16, 16)
                d16 = dstv[b][sl]
                l16 = d16 - lo
                ok = (l16 >= 0) & (l16 < HALF)
                d2 = 2 * l16
                dstA[b][sl] = jnp.where(ok, d2, SENT)
                if gather:
                    dstB[b][sl] = jnp.where(ok, d2 + 1, SENT)
                    s16 = srcv[b][sl]
                    sph = jnp.where(s16 >= HALF, s16 + (HALFP - HALF), s16)
                    s2 = 2 * sph
                    srcA[b][sl] = jnp.where(ok, s2, SENT)
                    srcB[b][sl] = jnp.where(ok, s2 + 1, SENT)

        def fire_gather(b):
            pltpu.async_copy(g_hbm.at[plsc.Indices(srcA[b],
                                                   ignored_value=SENT)],
                             rowsA[b], gsem[b])
            pltpu.async_copy(g_hbm.at[plsc.Indices(srcB[b],
                                                   ignored_value=SENT)],
                             rowsB[b], gsem[b])

        def wait_gather(b):
            pltpu.make_async_copy(g_hbm.at[pl.ds(0, CH)], rowsA[b],
                                  gsem[b]).wait()
            pltpu.make_async_copy(g_hbm.at[pl.ds(0, CH)], rowsB[b],
                                  gsem[b]).wait()

        def fire_scatter(b):
            pltpu.async_copy(rowsA[b],
                             acc.at[plsc.Indices(dstA[b], ignored_value=SENT)],
                             ssem[b], add=True)
            if gather:
                pltpu.async_copy(
                    rowsB[b],
                    acc.at[plsc.Indices(dstB[b], ignored_value=SENT)],
                    ssem[b], add=True)

        def wait_scatter(b):
            pltpu.make_async_copy(rowsA[b], acc.at[pl.ds(0, CH)],
                                  ssem[b]).wait()
            if gather:
                pltpu.make_async_copy(rowsB[b], acc.at[pl.ds(0, CH)],
                                      ssem[b]).wait()

        if gather:
            pltpu.sync_copy(g_hbm.at[pl.ds(c * (2 * HALFP) + s * RPT, RPT)],
                            acc.at[pl.ds(s * RPT, RPT)])
        else:
            pltpu.sync_copy(ones_hbm, acc.at[pl.ds(s * RPT, RPT)])
            for b in range(NB):
                pltpu.sync_copy(ones_hbm.at[pl.ds(0, CH)], rowsA[b])
        plsc.subcore_barrier()

        if gather:
            for b in range(NB):
                load_idx(b, b)
            for j in range(NB):
                b = j % NB
                wait_idx(b)
                compute_idx(b)
                fire_gather(b)
                load_idx(j + NB, b)
                if j > 0:
                    pb = (b + NB - 1) % NB
                    wait_gather(pb)
                    fire_scatter(pb)

            def group(g, carry):
                for b in range(NB):
                    wait_idx(b)
                    wait_scatter(b)
                    compute_idx(b)
                    fire_gather(b)
                    load_idx(g * NB + b + NB, b)
                    pb = (b + NB - 1) % NB
                    wait_gather(pb)
                    fire_scatter(pb)
                return carry

            lax.fori_loop(1, n_groups, group, 0)
            lastb = (n_chunks - 1) % NB
            wait_gather(lastb)
            fire_scatter(lastb)
            for b in range(NB):
                wait_scatter(b)
                pltpu.make_async_copy(dst_hbm.at[pl.ds(0, CH)], dstv[b],
                                      isem[b]).wait()
                pltpu.make_async_copy(src_hbm.at[pl.ds(0, CH)], srcv[b],
                                      isem[b]).wait()
        else:
            for b in range(NB):
                load_idx(b, b)
            for j in range(NB):
                b = j % NB
                wait_idx(b)
                compute_idx(b)
                fire_scatter(b)
                load_idx(j + NB, b)

            def group(g, carry):
                for b in range(NB):
                    wait_idx(b)
                    wait_scatter(b)
                    compute_idx(b)
                    fire_scatter(b)
                    load_idx(g * NB + b + NB, b)
                return carry

            lax.fori_loop(1, n_groups, group, 0)
            for b in range(NB):
                wait_scatter(b)
                pltpu.make_async_copy(dst_hbm.at[pl.ds(0, CH)], dstv[b],
                                      isem[b]).wait()
        plsc.subcore_barrier()
        pltpu.sync_copy(acc.at[pl.ds(s * RPT, RPT)],
                        out_hbm.at[pl.ds(c * (2 * HALFP) + s * RPT, RPT)])

    return prop


def _mm_plain_body(x_ref, w_ref, out_ref):
    out_ref[...] = jnp.dot(x_ref[...], w_ref[...],
                           preferred_element_type=jnp.float32)


_mm_plain = pl.pallas_call(
    _mm_plain_body,
    grid=(NROWS // _BR,),
    in_specs=[
        pl.BlockSpec((_BR, D), lambda i: (i, 0)),
        pl.BlockSpec((D, D), lambda i: (0, 0)),
    ],
    out_specs=pl.BlockSpec((_BR, D), lambda i: (i, 0)),
    out_shape=jax.ShapeDtypeStruct((NROWS, D), jnp.float32),
)


def _disscale_body(c_ref, deg_ref, g_ref, dis_ref):
    d0 = deg_ref[:, 0:1]
    dis = jnp.broadcast_to(lax.rsqrt(d0), (_BR, D))
    dis_ref[...] = dis
    g_ref[...] = c_ref[...] * dis


_disscale = pl.pallas_call(
    _disscale_body,
    grid=(NROWS // _BR,),
    in_specs=[
        pl.BlockSpec((_BR, D), lambda i: (i, 0)),
        pl.BlockSpec((_BR, D), lambda i: (i, 0)),
    ],
    out_specs=[
        pl.BlockSpec((_BR, D), lambda i: (i, 0)),
        pl.BlockSpec((_BR, D), lambda i: (i, 0)),
    ],
    out_shape=[
        jax.ShapeDtypeStruct((NROWS, D), jnp.float32),
        jax.ShapeDtypeStruct((NROWS, D), jnp.float32),
    ],
)


def _mm_mid(relu):
    def body(s_ref, dis_ref, b_ref, w_ref, out_ref):
        t = s_ref[...] * dis_ref[...] + b_ref[...]
        if relu:
            t = jnp.maximum(t, 0.0)
        o = jnp.dot(t, w_ref[...], preferred_element_type=jnp.float32)
        out_ref[...] = o * dis_ref[...]

    return pl.pallas_call(
        body,
        grid=(NROWS // _BR,),
        in_specs=[
            pl.BlockSpec((_BR, D), lambda i: (i, 0)),
            pl.BlockSpec((_BR, D), lambda i: (i, 0)),
            pl.BlockSpec((1, D), lambda i: (0, 0)),
            pl.BlockSpec((D, D), lambda i: (0, 0)),
        ],
        out_specs=pl.BlockSpec((_BR, D), lambda i: (i, 0)),
        out_shape=jax.ShapeDtypeStruct((NROWS, D), jnp.float32),
    )


def _finish_body(s_ref, dis_ref, b_ref, out_ref):
    out_ref[...] = s_ref[...] * dis_ref[...] + b_ref[...]


_finish = pl.pallas_call(
    _finish_body,
    grid=(NROWS // _BR,),
    in_specs=[
        pl.BlockSpec((_BR, D), lambda i: (i, 0)),
        pl.BlockSpec((_BR, D), lambda i: (i, 0)),
        pl.BlockSpec((1, D), lambda i: (0, 0)),
    ],
    out_specs=pl.BlockSpec((_BR, D), lambda i: (i, 0)),
    out_shape=jax.ShapeDtypeStruct((NROWS, D), jnp.float32),
)


def kernel(data, x, edge_index, W_e1, b_e1, W_e2, b_e2, W_d1, b_d1, W_d2, b_d2):
    src = edge_index[0]
    dst = edge_index[1]
    e = src.shape[0]
    epad = -(-e // (16 * CH * NB)) * (16 * CH * NB)
    if epad != e:
        pad = jnp.arange(epad - e, dtype=jnp.int32)
        src_p = jnp.concatenate([src, (pad * 2003) % N])
        dst_p = jnp.concatenate([dst, jnp.full((epad - e,), N, jnp.int32)])
    else:
        src_p, dst_p = src, dst

    prop_deg = _make_prop(epad, gather=False)
    prop = _make_prop(epad, gather=True)

    def run_prop(g):
        out2 = prop(g.reshape(2 * NROWS, HH), src_p, dst_p)
        return out2.reshape(NROWS, D)

    ones_blk = jnp.ones((RPT, HH), jnp.float32)
    deg2 = prop_deg(ones_blk, dst_p)

    zpad = jnp.zeros((HALFP - HALF, D), jnp.float32)
    xp = jnp.concatenate([x[:HALF], zpad, x[HALF:], zpad], axis=0)
    c1 = _mm_plain(xp, W_e1)
    g1, dis = _disscale(c1, deg2.reshape(NROWS, D))
    s1 = run_prop(g1)
    g2 = _mm_mid(True)(s1, dis, b_e1.reshape(1, D), W_e2)
    s2 = run_prop(g2)
    g3 = _mm_mid(False)(s2, dis, b_e2.reshape(1, D), W_d1)
    s3 = run_prop(g3)
    g4 = _mm_mid(True)(s3, dis, b_d1.reshape(1, D), W_d2)
    s4 = run_prop(g4)
    y = _finish(s4, dis, b_d2.reshape(1, D))

    x_ = jnp.concatenate([y[:HALF], y[HALFP:HALFP + HALF]], axis=0)
    return (x_, 1, 1)


# speedup vs baseline: 11.2926x; 1.0014x over previous
import functools

import jax
import jax.numpy as jnp
from jax import lax
from jax.experimental import pallas as pl
from jax.experimental.pallas import tpu as pltpu
from jax.experimental.pallas import tpu_sc as plsc

N = 10000
D = 256
HALF = 5000
HALFP = 5120
NROWS = 2 * HALFP
HH = 128
SENT = -1
ACC_ROWS = 10112
CH = 48
NB = 3
RPT = 632
CH16 = CH // 16
_BR = 1024


def _make_prop(epad, gather):
    e_per_tile = epad // 16
    n_chunks = e_per_tile // CH
    n_groups = n_chunks // NB
    mesh = plsc.VectorSubcoreMesh(core_axis_name="c", subcore_axis_name="s")

    def _vecs(shape, dt):
        return [pltpu.VMEM(shape, dt) for _ in range(NB)]

    scratch = (
        _vecs((CH,), jnp.int32)
        + _vecs((CH,), jnp.int32)
        + _vecs((CH,), jnp.int32)
        + _vecs((CH, HH), jnp.float32)
        + _vecs((CH, HH), jnp.float32)
        + [pltpu.VMEM_SHARED((ACC_ROWS, HH), jnp.float32)]
        + [pltpu.SemaphoreType.DMA] * (2 * NB)
    )
    if gather:
        scratch = (
            _vecs((CH,), jnp.int32)
            + _vecs((CH,), jnp.int32)
            + _vecs((CH,), jnp.int32)
            + scratch
            + [pltpu.SemaphoreType.DMA] * NB
        )

    @functools.partial(pl.kernel, mesh=mesh,
                       out_type=jax.ShapeDtypeStruct((2 * NROWS, HH),
                                                     jnp.float32),
                       compiler_params=pltpu.CompilerParams(
                           use_tc_tiling_on_sc=True),
                       scratch_types=scratch)
    def prop(*refs):
        if gather:
            g_hbm, src_hbm, dst_hbm, out_hbm = refs[:4]
            r = list(refs[4:])
            srcv, srcA, srcB = r[0:NB], r[NB:2*NB], r[2*NB:3*NB]
            r = r[3*NB:]
        else:
            ones_hbm, dst_hbm, out_hbm = refs[:3]
            r = list(refs[3:])
        dstv, dstA, dstB = r[0:NB], r[NB:2*NB], r[2*NB:3*NB]
        rowsA, rowsB = r[3*NB:4*NB], r[4*NB:5*NB]
        acc = r[5*NB]
        isem, ssem = r[5*NB+1:5*NB+1+NB], r[5*NB+1+NB:5*NB+1+2*NB]
        if gather:
            gsem = r[5*NB+1+2*NB:5*NB+1+3*NB]
        c = lax.axis_index("c")
        s = lax.axis_index("s")
        lo = c * HALF
        base = s * e_per_tile

        def load_idx(cj, b):
            eb = pl.multiple_of(base + lax.min(cj, n_chunks - 1) * CH, CH)
            pltpu.async_copy(dst_hbm.at[pl.ds(eb, CH)], dstv[b], isem[b])
            if gather:
                pltpu.async_copy(src_hbm.at[pl.ds(eb, CH)], srcv[b], isem[b])

        def wait_idx(b):
            pltpu.make_async_copy(dst_hbm.at[pl.ds(0, CH)], dstv[b],
                                  isem[b]).wait()
            if gather:
                pltpu.make_async_copy(src_hbm.at[pl.ds(0, CH)], srcv[b],
                                      isem[b]).wait()

        def compute_idx(b):
            for k in range(CH16):
                sl = pl.ds(k * 16, 16)
                d16 = dstv[b][sl]
                l16 = d16 - lo
                ok = (l16 >= 0) & (l16 < HALF)
                d2 = 2 * l16
                dstA[b][sl] = jnp.where(ok, d2, SENT)
                if gather:
                    dstB[b][sl] = jnp.where(ok, d2 + 1, SENT)
                    s16 = srcv[b][sl]
                    sph = jnp.where(s16 >= HALF, s16 + (HALFP - HALF), s16)
                    s2 = 2 * sph
                    srcA[b][sl] = jnp.where(ok, s2, SENT)
                    srcB[b][sl] = jnp.where(ok, s2 + 1, SENT)

        def fire_gather(b):
            pltpu.async_copy(g_hbm.at[plsc.Indices(srcA[b],
                                                   ignored_value=SENT)],
                             rowsA[b], gsem[b])
            pltpu.async_copy(g_hbm.at[plsc.Indices(srcB[b],
                                                   ignored_value=SENT)],
                             rowsB[b], gsem[b])

        def wait_gather(b):
            pltpu.make_async_copy(g_hbm.at[pl.ds(0, CH)], rowsA[b],
                                  gsem[b]).wait()
            pltpu.make_async_copy(g_hbm.at[pl.ds(0, CH)], rowsB[b],
                                  gsem[b]).wait()

        def fire_scatter(b):
            pltpu.async_copy(rowsA[b],
                             acc.at[plsc.Indices(dstA[b], ignored_value=SENT)],
                             ssem[b], add=True)
            if gather:
                pltpu.async_copy(
                    rowsB[b],
                    acc.at[plsc.Indices(dstB[b], ignored_value=SENT)],
                    ssem[b], add=True)

        def wait_scatter(b):
            pltpu.make_async_copy(rowsA[b], acc.at[pl.ds(0, CH)],
                                  ssem[b]).wait()
            if gather:
                pltpu.make_async_copy(rowsB[b], acc.at[pl.ds(0, CH)],
                                      ssem[b]).wait()

        if gather:
            pltpu.sync_copy(g_hbm.at[pl.ds(c * (2 * HALFP) + s * RPT, RPT)],
                            acc.at[pl.ds(s * RPT, RPT)])
        else:
            pltpu.sync_copy(ones_hbm, acc.at[pl.ds(s * RPT, RPT)])
            for b in range(NB):
                pltpu.sync_copy(ones_hbm.at[pl.ds(0, CH)], rowsA[b])
        plsc.subcore_barrier()

        if gather:
            for b in range(NB):
                load_idx(b, b)
            for j in range(NB):
                b = j % NB
                wait_idx(b)
                compute_idx(b)
                fire_gather(b)
                load_idx(j + NB, b)
                if j > 0:
                    pb = (b + NB - 1) % NB
                    wait_gather(pb)
                    fire_scatter(pb)

            def group(g, carry):
                for b in range(NB):
                    wait_idx(b)
                    wait_scatter(b)
                    compute_idx(b)
                    fire_gather(b)
                    load_idx(g * NB + b + NB, b)
                    pb = (b + NB - 1) % NB
                    wait_gather(pb)
                    fire_scatter(pb)
                return carry

            lax.fori_loop(1, n_groups, group, 0)
            lastb = (n_chunks - 1) % NB
            wait_gather(lastb)
            fire_scatter(lastb)
            for b in range(NB):
                wait_scatter(b)
                pltpu.make_async_copy(dst_hbm.at[pl.ds(0, CH)], dstv[b],
                                      isem[b]).wait()
                pltpu.make_async_copy(src_hbm.at[pl.ds(0, CH)], srcv[b],
                                      isem[b]).wait()
        else:
            for b in range(NB):
                load_idx(b, b)
            for j in range(NB):
                b = j % NB
                wait_idx(b)
                compute_idx(b)
                fire_scatter(b)
                load_idx(j + NB, b)

            def group(g, carry):
                for b in range(NB):
                    wait_idx(b)
                    wait_scatter(b)
                    compute_idx(b)
                    fire_scatter(b)
                    load_idx(g * NB + b + NB, b)
                return carry

            lax.fori_loop(1, n_groups, group, 0)
            for b in range(NB):
                wait_scatter(b)
                pltpu.make_async_copy(dst_hbm.at[pl.ds(0, CH)], dstv[b],
                                      isem[b]).wait()
        plsc.subcore_barrier()
        pltpu.sync_copy(acc.at[pl.ds(s * RPT, RPT)],
                        out_hbm.at[pl.ds(c * (2 * HALFP) + s * RPT, RPT)])

    return prop


def _mm_plain_body(x_ref, w_ref, out_ref):
    out_ref[...] = jnp.dot(x_ref[...], w_ref[...],
                           preferred_element_type=jnp.float32)


_mm_plain = pl.pallas_call(
    _mm_plain_body,
    grid=(NROWS // _BR,),
    in_specs=[
        pl.BlockSpec((_BR, D), lambda i: (i, 0)),
        pl.BlockSpec((D, D), lambda i: (0, 0)),
    ],
    out_specs=pl.BlockSpec((_BR, D), lambda i: (i, 0)),
    out_shape=jax.ShapeDtypeStruct((NROWS, D), jnp.float32),
)


def _disscale_body(c_ref, deg_ref, g_ref, dis_ref):
    d0 = deg_ref[:, 0:1]
    dis = jnp.broadcast_to(lax.rsqrt(d0), (_BR, D))
    dis_ref[...] = dis
    g_ref[...] = c_ref[...] * dis


_disscale = pl.pallas_call(
    _disscale_body,
    grid=(NROWS // _BR,),
    in_specs=[
        pl.BlockSpec((_BR, D), lambda i: (i, 0)),
        pl.BlockSpec((_BR, D), lambda i: (i, 0)),
    ],
    out_specs=[
        pl.BlockSpec((_BR, D), lambda i: (i, 0)),
        pl.BlockSpec((_BR, D), lambda i: (i, 0)),
    ],
    out_shape=[
        jax.ShapeDtypeStruct((NROWS, D), jnp.float32),
        jax.ShapeDtypeStruct((NROWS, D), jnp.float32),
    ],
)


def _mm_mid(relu):
    def body(s_ref, dis_ref, b_ref, w_ref, out_ref):
        t = s_ref[...] * dis_ref[...] + b_ref[...]
        if relu:
            t = jnp.maximum(t, 0.0)
        o = jnp.dot(t, w_ref[...], preferred_element_type=jnp.float32)
        out_ref[...] = o * dis_ref[...]

    return pl.pallas_call(
        body,
        grid=(NROWS // _BR,),
        in_specs=[
            pl.BlockSpec((_BR, D), lambda i: (i, 0)),
            pl.BlockSpec((_BR, D), lambda i: (i, 0)),
            pl.BlockSpec((1, D), lambda i: (0, 0)),
            pl.BlockSpec((D, D), lambda i: (0, 0)),
        ],
        out_specs=pl.BlockSpec((_BR, D), lambda i: (i, 0)),
        out_shape=jax.ShapeDtypeStruct((NROWS, D), jnp.float32),
    )


def _finish_body(s_ref, dis_ref, b_ref, out_ref):
    out_ref[...] = s_ref[...] * dis_ref[...] + b_ref[...]


_finish = pl.pallas_call(
    _finish_body,
    grid=(NROWS // _BR,),
    in_specs=[
        pl.BlockSpec((_BR, D), lambda i: (i, 0)),
        pl.BlockSpec((_BR, D), lambda i: (i, 0)),
        pl.BlockSpec((1, D), lambda i: (0, 0)),
    ],
    out_specs=pl.BlockSpec((_BR, D), lambda i: (i, 0)),
    out_shape=jax.ShapeDtypeStruct((NROWS, D), jnp.float32),
)


def kernel(data, x, edge_index, W_e1, b_e1, W_e2, b_e2, W_d1, b_d1, W_d2, b_d2):
    src = edge_index[0]
    dst = edge_index[1]
    e = src.shape[0]
    epad = -(-e // (16 * CH * NB)) * (16 * CH * NB)
    if epad != e:
        pad = jnp.arange(epad - e, dtype=jnp.int32)
        src_p = jnp.concatenate([src, (pad * 2003) % N])
        dst_p = jnp.concatenate([dst, jnp.full((epad - e,), N, jnp.int32)])
    else:
        src_p, dst_p = src, dst

    prop_deg = _make_prop(epad, gather=False)
    prop = _make_prop(epad, gather=True)

    def run_prop(g):
        out2 = prop(g.reshape(2 * NROWS, HH), src_p, dst_p)
        return out2.reshape(NROWS, D)

    ones_blk = jnp.ones((RPT, HH), jnp.float32)
    deg2 = prop_deg(ones_blk, dst_p)

    zpad = jnp.zeros((HALFP - HALF, D), jnp.float32)
    xp = jnp.concatenate([x[:HALF], zpad, x[HALF:], zpad], axis=0)
    c1 = _mm_plain(xp, W_e1)
    g1, dis = _disscale(c1, deg2.reshape(NROWS, D))
    s1 = run_prop(g1)
    g2 = _mm_mid(True)(s1, dis, b_e1.reshape(1, D), W_e2)
    s2 = run_prop(g2)
    g3 = _mm_mid(False)(s2, dis, b_e2.reshape(1, D), W_d1)
    s3 = run_prop(g3)
    g4 = _mm_mid(True)(s3, dis, b_d1.reshape(1, D), W_d2)
    s4 = run_prop(g4)
    y = _finish(s4, dis, b_d2.reshape(1, D))

    x_ = jnp.concatenate([y[:HALF], y[HALFP:HALFP + HALF]], axis=0)
    return (x_, 1, 1)
